# Initial kernel scaffold; baseline (speedup 1.0000x reference)
import jax
import jax.numpy as jnp
from jax.experimental import pallas as pl


def kernel(q, k_enc, k_actions, Wq_w, Wq_b, Wk_w, Wk_b, Wv_w, Wv_b):
    raise NotImplementedError("write your pallas kernel here")


# speedup vs baseline: 20.2875x; 20.2875x over previous
import functools
import math

import numpy as np
import jax
import jax.numpy as jnp
from jax.experimental import pallas as pl
from jax.experimental.pallas import tpu as pltpu

_TOPK = 33
_EPS = np.float32(1e-7)

_SIGN = np.int32(-2147483648)
_REST = np.int32(2147483647)


def _flip(f):
    b = jax.lax.bitcast_convert_type(f, jnp.int32)
    return jnp.where(b < 0, ~b, b | _SIGN)


def _unflip(u):
    b = jnp.where(u < 0, u & _REST, ~u)
    return jax.lax.bitcast_convert_type(b, jnp.float32)


def _proj_body(k_blk_ref, ka_ref, wkw_ref, wkb_ref, wvw_ref, wvb_ref,
               kpt_ref, vt_ref, *, na):
    kpt = jax.lax.dot_general(
        wkw_ref[...], k_blk_ref[...], (((1,), (1,)), ((), ())),
        preferred_element_type=jnp.float32)
    kpt_ref[...] = kpt + wkb_ref[...]
    ka = ka_ref[...]
    iota = jax.lax.broadcasted_iota(jnp.int32, (na, ka.shape[1]), 0)
    onehot_t = (iota == ka).astype(jnp.float32)
    vt = jax.lax.dot_general(
        wvw_ref[...], onehot_t, (((1,), (0,)), ((), ())),
        preferred_element_type=jnp.float32)
    vt_ref[...] = vt + wvb_ref[...]


def _attn_body(q_ref, wqw_ref, wqb_ref, kpt_ref, vt_ref,
               sparse_ref, final_ref, *, inv_temp, topk):
    qp = jax.lax.dot_general(
        q_ref[...], wqw_ref[...], (((1,), (1,)), ((), ())),
        preferred_element_type=jnp.float32)
    qp = (qp + wqb_ref[...]) * inv_temp
    s = jax.lax.dot_general(
        qp, kpt_ref[...], (((1,), (0,)), ((), ())),
        preferred_element_type=jnp.float32)
    sparse_ref[...] = s

    rmax = jnp.max(s, axis=1, keepdims=True)
    rmin = jnp.min(s, axis=1, keepdims=True)
    lo0 = _flip(rmin)
    hi0 = _flip(rmax)
    kcnt = np.float32(topk)

    def step(_, carry):
        lo, hi = carry
        d = hi - lo
        half = jax.lax.shift_right_logical(d + 1, 1)
        mid = lo + half
        mid_f = _unflip(mid)
        cnt = jnp.sum((sparse_ref[...] >= mid_f).astype(jnp.float32),
                      axis=1, keepdims=True)
        ge = cnt >= kcnt
        return jnp.where(ge, mid, lo), jnp.where(ge, hi, mid - 1)

    lo, _ = jax.lax.fori_loop(0, 32, step, (lo0, hi0))
    delta = _unflip(lo) + _EPS

    w_sum = jnp.sum(jnp.maximum(sparse_ref[...] - delta, 0.0),
                    axis=1, keepdims=True) + _EPS
    out = jnp.maximum(sparse_ref[...] - delta, 0.0) / w_sum
    sparse_ref[...] = out
    final_ref[...] = jax.lax.dot_general(
        out, vt_ref[...], (((1,), (1,)), ((), ())),
        preferred_element_type=jnp.float32)


@jax.jit
def kernel(q, k_enc, k_actions, Wq_w, Wq_b, Wk_w, Wk_b, Wv_w, Wv_b):
    nq, dm = q.shape
    nk = k_enc.shape[0]
    dk = Wq_w.shape[0]
    na = Wv_w.shape[0]
    inv_temp = np.float32(1.0 / math.sqrt(dk))

    bk = min(2048, nk)
    bq = min(64, nq)

    ka2 = k_actions.astype(jnp.int32).reshape(1, nk)
    wkb = Wk_b.reshape(dk, 1)
    wvb = Wv_b.reshape(na, 1)
    wqb = Wq_b.reshape(1, dk)

    kpt, vt = pl.pallas_call(
        functools.partial(_proj_body, na=na),
        grid=(nk // bk,),
        in_specs=[
            pl.BlockSpec((bk, dm), lambda j: (j, 0)),
            pl.BlockSpec((1, bk), lambda j: (0, j)),
            pl.BlockSpec((dk, dm), lambda j: (0, 0)),
            pl.BlockSpec((dk, 1), lambda j: (0, 0)),
            pl.BlockSpec((na, na), lambda j: (0, 0)),
            pl.BlockSpec((na, 1), lambda j: (0, 0)),
        ],
        out_specs=[
            pl.BlockSpec((dk, bk), lambda j: (0, j)),
            pl.BlockSpec((na, bk), lambda j: (0, j)),
        ],
        out_shape=[
            jax.ShapeDtypeStruct((dk, nk), jnp.float32),
            jax.ShapeDtypeStruct((na, nk), jnp.float32),
        ],
    )(k_enc, ka2, Wk_w, wkb, Wv_w, wvb)

    sparse, final = pl.pallas_call(
        functools.partial(_attn_body, inv_temp=inv_temp, topk=_TOPK),
        grid=(nq // bq,),
        in_specs=[
            pl.BlockSpec((bq, dm), lambda i: (i, 0)),
            pl.BlockSpec((dk, dm), lambda i: (0, 0)),
            pl.BlockSpec((1, dk), lambda i: (0, 0)),
            pl.BlockSpec((dk, nk), lambda i: (0, 0)),
            pl.BlockSpec((na, nk), lambda i: (0, 0)),
        ],
        out_specs=[
            pl.BlockSpec((bq, nk), lambda i: (i, 0)),
            pl.BlockSpec((bq, na), lambda i: (i, 0)),
        ],
        out_shape=[
            jax.ShapeDtypeStruct((nq, nk), jnp.float32),
            jax.ShapeDtypeStruct((nq, na), jnp.float32),
        ],
    )(q, Wq_w, wqb, kpt, vt)

    return final, sparse
